# baseline (speedup 1.0000x reference)
import jax
import jax.numpy as jnp
from jax import lax
from jax.experimental import pallas as pl
from jax.experimental.pallas import tpu as pltpu
from jax.experimental.pallas import tpu_sc as plsc


NC, NS = 2, 16
NW = NC * NS
L = 16


def _sc_add_body(x_hbm, t_hbm, o_hbm, table_v, buf_v):
    wid = lax.axis_index("s") * NC + lax.axis_index("c")
    rows = x_hbm.shape[0] // NW
    base = wid * rows
    embed, seq = t_hbm.shape
    pltpu.sync_copy(t_hbm, table_v)

    def row_body(b, carry):
        pltpu.sync_copy(x_hbm.at[base + b], buf_v)

        def step(r, c):
            for u in range(seq // L):
                o = u * L
                buf_v[r, pl.ds(o, L)] = (
                    buf_v[r, pl.ds(o, L)] + table_v[r, pl.ds(o, L)]
                )
            return c

        lax.fori_loop(0, embed, step, 0)
        pltpu.sync_copy(buf_v, o_hbm.at[base + b])
        return carry

    lax.fori_loop(0, rows, row_body, 0)


def kernel(x, pos_table):
    batch, seq_len, embed = x.shape
    xt = jnp.transpose(x, (0, 2, 1))
    tt = jnp.transpose(pos_table[:seq_len], (1, 0))
    mesh = plsc.VectorSubcoreMesh(core_axis_name="c", subcore_axis_name="s")
    out_t = pl.kernel(
        _sc_add_body,
        out_type=jax.ShapeDtypeStruct(xt.shape, x.dtype),
        mesh=mesh,
        scratch_types=[
            pltpu.VMEM((embed, seq_len), jnp.float32),
            pltpu.VMEM((embed, seq_len), jnp.float32),
        ],
        compiler_params=pltpu.CompilerParams(use_tc_tiling_on_sc=True),
    )(xt, tt)
    return jnp.transpose(out_t, (0, 2, 1))


# speedup vs baseline: 1.5679x; 1.5679x over previous
import jax
import jax.numpy as jnp
from jax import lax
from jax.experimental import pallas as pl
from jax.experimental.pallas import tpu as pltpu
from jax.experimental.pallas import tpu_sc as plsc


NC, NS = 2, 16
NW = NC * NS
L = 16
SC_ROWS = 320
BATCH_BLOCK = 64


def _sc_add_body(x_hbm, t_hbm, o_hbm, table_v, buf_v):
    wid = lax.axis_index("s") * NC + lax.axis_index("c")
    rows = o_hbm.shape[0] // NW
    base = wid * rows
    embed, seq = t_hbm.shape
    pltpu.sync_copy(t_hbm, table_v)

    def row_body(b, carry):
        pltpu.sync_copy(x_hbm.at[base + b], buf_v)

        def step(r, c):
            for u in range(seq // L):
                o = u * L
                buf_v[r, pl.ds(o, L)] = (
                    buf_v[r, pl.ds(o, L)] + table_v[r, pl.ds(o, L)]
                )
            return c

        lax.fori_loop(0, embed, step, 0)
        pltpu.sync_copy(buf_v, o_hbm.at[base + b])
        return carry

    lax.fori_loop(0, rows, row_body, 0)


def _tc_add_body(x_ref, t_ref, o_ref):
    o_ref[...] = x_ref[...] + t_ref[...][None, :, :]


def kernel(x, pos_table):
    batch, seq_len, embed = x.shape
    xt = jnp.transpose(x, (0, 2, 1))
    tt = jnp.transpose(pos_table[:seq_len], (1, 0))

    mesh = plsc.VectorSubcoreMesh(core_axis_name="c", subcore_axis_name="s")
    sc_part = pl.kernel(
        _sc_add_body,
        out_type=jax.ShapeDtypeStruct((SC_ROWS, embed, seq_len), x.dtype),
        mesh=mesh,
        scratch_types=[
            pltpu.VMEM((embed, seq_len), jnp.float32),
            pltpu.VMEM((embed, seq_len), jnp.float32),
        ],
        compiler_params=pltpu.CompilerParams(use_tc_tiling_on_sc=True),
    )(xt, tt)

    blk0 = SC_ROWS // BATCH_BLOCK
    tc_full = pl.pallas_call(
        _tc_add_body,
        grid=((batch - SC_ROWS) // BATCH_BLOCK,),
        in_specs=[
            pl.BlockSpec(
                (BATCH_BLOCK, embed, seq_len), lambda i: (i + blk0, 0, 0)
            ),
            pl.BlockSpec((embed, seq_len), lambda i: (0, 0)),
        ],
        out_specs=pl.BlockSpec(
            (BATCH_BLOCK, embed, seq_len), lambda i: (i + blk0, 0, 0)
        ),
        out_shape=jax.ShapeDtypeStruct((batch, embed, seq_len), x.dtype),
    )(xt, tt)

    out_t = lax.dynamic_update_slice(tc_full, sc_part, (0, 0, 0))
    return jnp.transpose(out_t, (0, 2, 1))
